# baseline (speedup 1.0000x reference)
import numpy as np
import jax
import jax.numpy as jnp
from jax.experimental import pallas as pl
from jax.experimental.pallas import tpu as pltpu


def _banded_reflect(weights, n):
    m = np.zeros((n, n), np.float32)
    for i in range(n):
        for d, wt in enumerate(weights):
            if wt == 0.0:
                continue
            idx = i + d - 2
            if idx < 0:
                idx = -idx
            elif idx >= n:
                idx = 2 * n - 2 - idx
            m[i, idx] += wt
    return m


def _split_bf16(t, terms):
    parts = []
    r = t
    for _ in range(terms):
        p = r.astype(jnp.bfloat16)
        parts.append(p)
        r = r - p.astype(jnp.float32)
    return parts


def _rowmat_f32(m_ref, t, terms):
    out = None
    for p in _split_bf16(t, terms):
        term = jax.lax.dot_general(m_ref[...], p, (((1,), (0,)), ((), ())),
                                   preferred_element_type=jnp.float32)
        out = term if out is None else out + term
    return out


def kernel(x):
    b, s, c, h, w = x.shape
    gk = (0.0625, 0.25, 0.375, 0.25, 0.0625)
    sm = (1.0, 4.0, 6.0, 4.0, 1.0)
    d2 = (1.0, 0.0, -2.0, 0.0, 1.0)

    g_col = jnp.asarray(_banded_reflect(gk, w).T, jnp.bfloat16)
    g_row = jnp.asarray(_banded_reflect(gk, h), jnp.bfloat16)
    l_col = jnp.asarray(
        np.concatenate([_banded_reflect(d2, w).T,
                        _banded_reflect(sm, w).T], axis=1), jnp.bfloat16)
    s_row = jnp.asarray(_banded_reflect(sm, h), jnp.bfloat16)
    d_row = jnp.asarray(_banded_reflect(d2, h), jnp.bfloat16)

    def body(x_ref, gc_ref, gr_ref, lc_ref, sr_ref, dr_ref, o_ref, best_ref):
        si = pl.program_id(1)
        img = (x_ref[0, 0, 0] + x_ref[0, 0, 1] + x_ref[0, 0, 2]) * (1.0 / 3.0)
        imgb = img.astype(jnp.bfloat16)

        u = jax.lax.dot_general(imgb, gc_ref[...], (((1,), (0,)), ((), ())),
                                preferred_element_type=jnp.float32)
        blur = _rowmat_f32(gr_ref, u, 3)

        blurb = blur.astype(jnp.bfloat16)
        v = jax.lax.dot_general(blurb, lc_ref[...], (((1,), (0,)), ((), ())),
                                preferred_element_type=jnp.float32)
        lap = (_rowmat_f32(sr_ref, v[:, 0:w], 2) +
               _rowmat_f32(dr_ref, v[:, w:2 * w], 2))

        @pl.when(si == 0)
        def _init():
            best_ref[...] = lap
            for ci in range(c):
                o_ref[0, ci] = x_ref[0, 0, ci]

        @pl.when(si > 0)
        def _update():
            prev = best_ref[...]
            pred = lap > prev
            best_ref[...] = jnp.where(pred, lap, prev)
            for ci in range(c):
                o_ref[0, ci] = jnp.where(pred, x_ref[0, 0, ci], o_ref[0, ci])

    return pl.pallas_call(
        body,
        grid=(b, s),
        in_specs=[
            pl.BlockSpec((1, 1, c, h, w), lambda i, j: (i, j, 0, 0, 0)),
            pl.BlockSpec((w, w), lambda i, j: (0, 0)),
            pl.BlockSpec((h, h), lambda i, j: (0, 0)),
            pl.BlockSpec((w, 2 * w), lambda i, j: (0, 0)),
            pl.BlockSpec((h, h), lambda i, j: (0, 0)),
            pl.BlockSpec((h, h), lambda i, j: (0, 0)),
        ],
        out_specs=pl.BlockSpec((1, c, h, w), lambda i, j: (i, 0, 0, 0)),
        out_shape=jax.ShapeDtypeStruct((b, c, h, w), x.dtype),
        scratch_shapes=[pltpu.VMEM((h, w), jnp.float32)],
        compiler_params=pltpu.CompilerParams(
            dimension_semantics=("parallel", "arbitrary")),
    )(x, g_col, g_row, l_col, s_row, d_row)


# speedup vs baseline: 1.0400x; 1.0400x over previous
import numpy as np
import jax
import jax.numpy as jnp
from jax.experimental import pallas as pl
from jax.experimental.pallas import tpu as pltpu


def _banded_reflect(weights, n):
    m = np.zeros((n, n), np.float32)
    for i in range(n):
        for d, wt in enumerate(weights):
            if wt == 0.0:
                continue
            idx = i + d - 2
            if idx < 0:
                idx = -idx
            elif idx >= n:
                idx = 2 * n - 2 - idx
            m[i, idx] += wt
    return m


def _split_bf16(t, terms):
    parts = []
    r = t
    for _ in range(terms):
        p = r.astype(jnp.bfloat16)
        parts.append(p)
        r = r - p.astype(jnp.float32)
    return parts


def _rowmat_f32(m_ref, t, terms):
    out = None
    for p in _split_bf16(t, terms):
        term = jax.lax.dot_general(m_ref[...], p, (((1,), (0,)), ((), ())),
                                   preferred_element_type=jnp.float32)
        out = term if out is None else out + term
    return out


def kernel(x):
    b, s, c, h, w = x.shape
    gk = (0.0625, 0.25, 0.375, 0.25, 0.0625)
    sm = (1.0, 4.0, 6.0, 4.0, 1.0)
    d2 = (1.0, 0.0, -2.0, 0.0, 1.0)

    g_col = jnp.asarray(_banded_reflect(gk, w).T, jnp.bfloat16)
    g_row = jnp.asarray(_banded_reflect(gk, h), jnp.bfloat16)
    l_col = jnp.asarray(
        np.concatenate([_banded_reflect(d2, w).T,
                        _banded_reflect(sm, w).T], axis=1), jnp.bfloat16)
    s_row = jnp.asarray(_banded_reflect(sm, h), jnp.bfloat16)
    d_row = jnp.asarray(_banded_reflect(d2, h), jnp.bfloat16)

    f_per_step = 4

    def body(x_ref, gc_ref, gr_ref, lc_ref, sr_ref, dr_ref, o_ref, best_ref):
        si = pl.program_id(1)

        imgbs = [
            ((x_ref[0, f, 0] + x_ref[0, f, 1] + x_ref[0, f, 2]) *
             (1.0 / 3.0)).astype(jnp.bfloat16) for f in range(f_per_step)
        ]
        us = [
            jax.lax.dot_general(im, gc_ref[...], (((1,), (0,)), ((), ())),
                                preferred_element_type=jnp.float32)
            for im in imgbs
        ]
        blurbs = [_rowmat_f32(gr_ref, u, 3).astype(jnp.bfloat16) for u in us]
        vs = [
            jax.lax.dot_general(bb, lc_ref[...], (((1,), (0,)), ((), ())),
                                preferred_element_type=jnp.float32)
            for bb in blurbs
        ]
        laps = [(_rowmat_f32(sr_ref, v[:, 0:w], 2) +
                 _rowmat_f32(dr_ref, v[:, w:2 * w], 2)) for v in vs]

        def update(f, prev):
            pred = laps[f] > prev
            for ci in range(c):
                o_ref[0, ci] = jnp.where(pred, x_ref[0, f, ci], o_ref[0, ci])
            return jnp.where(pred, laps[f], prev)

        @pl.when(si == 0)
        def _init():
            best = laps[0]
            for ci in range(c):
                o_ref[0, ci] = x_ref[0, 0, ci]
            for f in range(1, f_per_step):
                best = update(f, best)
            best_ref[...] = best

        @pl.when(si > 0)
        def _update():
            best = best_ref[...]
            for f in range(f_per_step):
                best = update(f, best)
            best_ref[...] = best

    return pl.pallas_call(
        body,
        grid=(b, s // f_per_step),
        in_specs=[
            pl.BlockSpec((1, f_per_step, c, h, w),
                         lambda i, j: (i, j, 0, 0, 0)),
            pl.BlockSpec((w, w), lambda i, j: (0, 0)),
            pl.BlockSpec((h, h), lambda i, j: (0, 0)),
            pl.BlockSpec((w, 2 * w), lambda i, j: (0, 0)),
            pl.BlockSpec((h, h), lambda i, j: (0, 0)),
            pl.BlockSpec((h, h), lambda i, j: (0, 0)),
        ],
        out_specs=pl.BlockSpec((1, c, h, w), lambda i, j: (i, 0, 0, 0)),
        out_shape=jax.ShapeDtypeStruct((b, c, h, w), x.dtype),
        scratch_shapes=[pltpu.VMEM((h, w), jnp.float32)],
        compiler_params=pltpu.CompilerParams(
            dimension_semantics=("parallel", "arbitrary")),
    )(x, g_col, g_row, l_col, s_row, d_row)


# speedup vs baseline: 1.2737x; 1.2247x over previous
import numpy as np
import jax
import jax.numpy as jnp
from jax.experimental import pallas as pl
from jax.experimental.pallas import tpu as pltpu


def _banded_reflect_colmat(weights, n):
    m = np.zeros((n, n), np.float32)
    for j in range(n):
        for d, wt in enumerate(weights):
            if wt == 0.0:
                continue
            idx = j + d - 2
            if idx < 0:
                idx = -idx
            elif idx >= n:
                idx = 2 * n - 2 - idx
            m[idx, j] += wt
    return m


def _pad_rows_reflect2(a, h):
    return jnp.concatenate(
        [a[2:3], a[1:2], a, a[h - 2:h - 1], a[h - 3:h - 2]], axis=0)


def _row_sym5(p, w0, w1, w2, h):
    return (w0 * (p[0:h] + p[4:h + 4]) + w1 * (p[1:h + 1] + p[3:h + 3]) +
            w2 * p[2:h + 2])


def _row_d2(p, h):
    return (p[0:h] + p[4:h + 4]) - 2.0 * p[2:h + 2]


def kernel(x):
    b, s, c, h, w = x.shape
    gk = (0.0625, 0.25, 0.375, 0.25, 0.0625)
    sm = (1.0, 4.0, 6.0, 4.0, 1.0)
    d2 = (1.0, 0.0, -2.0, 0.0, 1.0)

    gmat = jnp.asarray(_banded_reflect_colmat(gk, w), jnp.bfloat16)
    lmat = jnp.asarray(
        np.concatenate([_banded_reflect_colmat(d2, w),
                        _banded_reflect_colmat(sm, w)], axis=1), jnp.bfloat16)

    f_per_step = 2

    def body(x_ref, g_ref, l_ref, o_ref, best_ref):
        si = pl.program_id(1)

        imgbs = [
            ((x_ref[0, f, 0] + x_ref[0, f, 1] + x_ref[0, f, 2]) *
             (1.0 / 3.0)).astype(jnp.bfloat16) for f in range(f_per_step)
        ]
        ts = [
            jax.lax.dot_general(im, g_ref[...], (((1,), (0,)), ((), ())),
                                preferred_element_type=jnp.float32)
            for im in imgbs
        ]
        blurbs = [
            _row_sym5(_pad_rows_reflect2(t, h), gk[0], gk[1], gk[2],
                      h).astype(jnp.bfloat16) for t in ts
        ]
        vs = [
            jax.lax.dot_general(bb, l_ref[...], (((1,), (0,)), ((), ())),
                                preferred_element_type=jnp.float32)
            for bb in blurbs
        ]
        laps = [
            _row_sym5(_pad_rows_reflect2(v[:, 0:w], h), sm[0], sm[1], sm[2],
                      h) + _row_d2(_pad_rows_reflect2(v[:, w:2 * w], h), h)
            for v in vs
        ]

        def update(f, prev):
            pred = laps[f] > prev
            for ci in range(c):
                o_ref[0, ci] = jnp.where(pred, x_ref[0, f, ci], o_ref[0, ci])
            return jnp.where(pred, laps[f], prev)

        @pl.when(si == 0)
        def _init():
            best = laps[0]
            for ci in range(c):
                o_ref[0, ci] = x_ref[0, 0, ci]
            for f in range(1, f_per_step):
                best = update(f, best)
            best_ref[...] = best

        @pl.when(si > 0)
        def _update():
            best = best_ref[...]
            for f in range(f_per_step):
                best = update(f, best)
            best_ref[...] = best

    return pl.pallas_call(
        body,
        grid=(b, s // f_per_step),
        in_specs=[
            pl.BlockSpec((1, f_per_step, c, h, w),
                         lambda i, j: (i, j, 0, 0, 0)),
            pl.BlockSpec((w, w), lambda i, j: (0, 0)),
            pl.BlockSpec((w, 2 * w), lambda i, j: (0, 0)),
        ],
        out_specs=pl.BlockSpec((1, c, h, w), lambda i, j: (i, 0, 0, 0)),
        out_shape=jax.ShapeDtypeStruct((b, c, h, w), x.dtype),
        scratch_shapes=[pltpu.VMEM((h, w), jnp.float32)],
        compiler_params=pltpu.CompilerParams(
            dimension_semantics=("parallel", "arbitrary")),
    )(x, gmat, lmat)


# speedup vs baseline: 1.4173x; 1.1128x over previous
import numpy as np
import jax
import jax.numpy as jnp
from jax.experimental import pallas as pl
from jax.experimental.pallas import tpu as pltpu


def _banded_reflect_colmat(weights, n):
    m = np.zeros((n, n), np.float32)
    for j in range(n):
        for d, wt in enumerate(weights):
            if wt == 0.0:
                continue
            idx = j + d - 2
            if idx < 0:
                idx = -idx
            elif idx >= n:
                idx = 2 * n - 2 - idx
            m[idx, j] += wt
    return m


def _pad_rows_reflect2(a, h):
    return jnp.concatenate(
        [a[2:3], a[1:2], a, a[h - 2:h - 1], a[h - 3:h - 2]], axis=0)


def _row_sym5(p, w0, w1, w2, h):
    return (w0 * (p[0:h] + p[4:h + 4]) + w1 * (p[1:h + 1] + p[3:h + 3]) +
            w2 * p[2:h + 2])


def _row_d2(p, h):
    return (p[0:h] + p[4:h + 4]) - 2.0 * p[2:h + 2]


def kernel(x):
    b, s, c, h, w = x.shape
    gk = (0.0625, 0.25, 0.375, 0.25, 0.0625)
    sm = (1.0, 4.0, 6.0, 4.0, 1.0)

    gmat = jnp.asarray(_banded_reflect_colmat(gk, w), jnp.bfloat16)
    lmat = jnp.asarray(
        np.concatenate(
            [_banded_reflect_colmat((1.0, 0.0, -2.0, 0.0, 1.0), w),
             _banded_reflect_colmat(sm, w)], axis=1), jnp.bfloat16)

    f_per_step = 2

    def body(x_ref, g_ref, l_ref, o_ref, best_ref):
        si = pl.program_id(1)

        imgbps = [
            _pad_rows_reflect2(
                ((x_ref[0, f, 0] + x_ref[0, f, 1] + x_ref[0, f, 2]) *
                 (1.0 / 3.0)).astype(jnp.bfloat16), h)
            for f in range(f_per_step)
        ]
        tps = [
            jax.lax.dot_general(im, g_ref[...], (((1,), (0,)), ((), ())),
                                preferred_element_type=jnp.float32)
            for im in imgbps
        ]
        blurbps = [
            _pad_rows_reflect2(
                _row_sym5(tp, gk[0], gk[1], gk[2], h).astype(jnp.bfloat16), h)
            for tp in tps
        ]
        vps = [
            jax.lax.dot_general(bb, l_ref[...], (((1,), (0,)), ((), ())),
                                preferred_element_type=jnp.float32)
            for bb in blurbps
        ]
        laps = [
            _row_sym5(vp[:, 0:w], sm[0], sm[1], sm[2], h) +
            _row_d2(vp[:, w:2 * w], h) for vp in vps
        ]

        def update(f, prev):
            pred = laps[f] > prev
            for ci in range(c):
                o_ref[0, ci] = jnp.where(pred, x_ref[0, f, ci], o_ref[0, ci])
            return jnp.where(pred, laps[f], prev)

        @pl.when(si == 0)
        def _init():
            best = laps[0]
            for ci in range(c):
                o_ref[0, ci] = x_ref[0, 0, ci]
            for f in range(1, f_per_step):
                best = update(f, best)
            best_ref[...] = best

        @pl.when(si > 0)
        def _update():
            best = best_ref[...]
            for f in range(f_per_step):
                best = update(f, best)
            best_ref[...] = best

    return pl.pallas_call(
        body,
        grid=(b, s // f_per_step),
        in_specs=[
            pl.BlockSpec((1, f_per_step, c, h, w),
                         lambda i, j: (i, j, 0, 0, 0)),
            pl.BlockSpec((w, w), lambda i, j: (0, 0)),
            pl.BlockSpec((w, 2 * w), lambda i, j: (0, 0)),
        ],
        out_specs=pl.BlockSpec((1, c, h, w), lambda i, j: (i, 0, 0, 0)),
        out_shape=jax.ShapeDtypeStruct((b, c, h, w), x.dtype),
        scratch_shapes=[pltpu.VMEM((h, w), jnp.float32)],
        compiler_params=pltpu.CompilerParams(
            dimension_semantics=("parallel", "arbitrary")),
    )(x, gmat, lmat)


# speedup vs baseline: 1.4460x; 1.0203x over previous
import numpy as np
import jax
import jax.numpy as jnp
from jax.experimental import pallas as pl
from jax.experimental.pallas import tpu as pltpu


def _banded_reflect_colmat(weights, n):
    m = np.zeros((n, n), np.float32)
    for j in range(n):
        for d, wt in enumerate(weights):
            if wt == 0.0:
                continue
            idx = j + d - 2
            if idx < 0:
                idx = -idx
            elif idx >= n:
                idx = 2 * n - 2 - idx
            m[idx, j] += wt
    return m


def _pad_rows_reflect2(a, h):
    return jnp.concatenate(
        [a[2:3], a[1:2], a, a[h - 2:h - 1], a[h - 3:h - 2]], axis=0)


def _row_sym5(p, w0, w1, w2, h):
    return (w0 * (p[0:h] + p[4:h + 4]) + w1 * (p[1:h + 1] + p[3:h + 3]) +
            w2 * p[2:h + 2])


def _row_d2(p, h):
    return (p[0:h] + p[4:h + 4]) - 2.0 * p[2:h + 2]


def kernel(x):
    b, s, c, h, w = x.shape
    gk = (0.0625, 0.25, 0.375, 0.25, 0.0625)
    sm = (1.0, 4.0, 6.0, 4.0, 1.0)

    gmat = jnp.asarray(_banded_reflect_colmat(gk, w), jnp.bfloat16)
    lmat = jnp.asarray(
        np.concatenate(
            [_banded_reflect_colmat((1.0, 0.0, -2.0, 0.0, 1.0), w),
             _banded_reflect_colmat(sm, w)], axis=1), jnp.bfloat16)

    f_per_step = 4

    def body(x_ref, g_ref, l_ref, o_ref, best_ref):
        si = pl.program_id(1)

        imgbps = [
            _pad_rows_reflect2(
                ((x_ref[0, f, 0] + x_ref[0, f, 1] + x_ref[0, f, 2]) *
                 (1.0 / 3.0)).astype(jnp.bfloat16), h)
            for f in range(f_per_step)
        ]
        tps = [
            jax.lax.dot_general(im, g_ref[...], (((1,), (0,)), ((), ())),
                                preferred_element_type=jnp.float32)
            for im in imgbps
        ]
        blurbps = [
            _pad_rows_reflect2(
                _row_sym5(tp, gk[0], gk[1], gk[2], h).astype(jnp.bfloat16), h)
            for tp in tps
        ]
        vps = [
            jax.lax.dot_general(bb, l_ref[...], (((1,), (0,)), ((), ())),
                                preferred_element_type=jnp.float32)
            for bb in blurbps
        ]
        laps = [
            _row_sym5(vp[:, 0:w], sm[0], sm[1], sm[2], h) +
            _row_d2(vp[:, w:2 * w], h) for vp in vps
        ]

        def pair(a, b):
            lap_a, get_a = a
            lap_b, get_b = b
            pick = lap_b > lap_a
            return (jnp.where(pick, lap_b, lap_a),
                    lambda ci: jnp.where(pick, get_b(ci), get_a(ci)))

        entrants = [(laps[f], lambda ci, f=f: x_ref[0, f, ci])
                    for f in range(f_per_step)]
        while len(entrants) > 1:
            entrants = [pair(entrants[i], entrants[i + 1])
                        for i in range(0, len(entrants), 2)]
        lap_m, get_m = entrants[0]

        @pl.when(si == 0)
        def _init():
            best_ref[...] = lap_m
            for ci in range(c):
                o_ref[0, ci] = get_m(ci)

        @pl.when(si > 0)
        def _update():
            pred = lap_m > best_ref[...]
            best_ref[...] = jnp.where(pred, lap_m, best_ref[...])
            for ci in range(c):
                o_ref[0, ci] = jnp.where(pred, get_m(ci), o_ref[0, ci])

    return pl.pallas_call(
        body,
        grid=(b, s // f_per_step),
        in_specs=[
            pl.BlockSpec((1, f_per_step, c, h, w),
                         lambda i, j: (i, j, 0, 0, 0)),
            pl.BlockSpec((w, w), lambda i, j: (0, 0)),
            pl.BlockSpec((w, 2 * w), lambda i, j: (0, 0)),
        ],
        out_specs=pl.BlockSpec((1, c, h, w), lambda i, j: (i, 0, 0, 0)),
        out_shape=jax.ShapeDtypeStruct((b, c, h, w), x.dtype),
        scratch_shapes=[pltpu.VMEM((h, w), jnp.float32)],
        compiler_params=pltpu.CompilerParams(
            dimension_semantics=("parallel", "arbitrary")),
    )(x, gmat, lmat)
